# baseline (speedup 1.0000x reference)
import jax
import jax.numpy as jnp
from jax import lax
from jax.experimental import pallas as pl
from jax.experimental.pallas import tpu as pltpu
from jax.experimental.pallas import tpu_sc as plsc

_SIZE = 220000
_EMB = 256
_BATCH = 16384
_MOM = 0.5

_NC, _NS, _L = 2, 16, 16
_NW = _NC * _NS
_RPW = 6880
_PPAD = 6960
_CH = 16
_NIV = _BATCH // _L
_NWV = _RPW // _L

_MESH = plsc.VectorSubcoreMesh(core_axis_name="c", subcore_axis_name="s",
                               num_cores=_NC, num_subcores=_NS)
_PARAMS = pltpu.CompilerParams(needs_layout_passes=False)


def _worker_id():
    return lax.axis_index("c") * _NS + lax.axis_index("s")


def _body_a(mem_in, k_in, idx_in, vals_out, rows_out, pos_out, cnt_out,
            idx_v, win_v, rows_v, pos_v, cnt_v,
            kbuf0, kbuf1, oldbuf0, oldbuf1, sbuf0, sbuf1,
            sem_g0, sem_g1, sem_s0, sem_s1):
    wid = _worker_id()
    base = wid * _RPW

    pltpu.sync_copy(idx_in, idx_v)

    iota = lax.iota(jnp.int32, _L)
    neg1 = jnp.full((_L,), -1, jnp.int32)

    @pl.loop(0, _NWV, unroll=5)
    def _init(g):
        win_v[pl.ds(g * _L, _L)] = neg1

    @pl.loop(0, _NIV, unroll=4)
    def _scan(g):
        vi = idx_v[pl.ds(g * _L, _L)]
        m = (vi >= base) & (vi < base + _RPW)
        local = jnp.clip(vi - base, 0, _RPW - 1)
        pos = g * _L + iota
        plsc.store_scatter(win_v, [local], pos, mask=m)

    @pl.loop(0, _NIV, unroll=4)
    def _fix(g):
        vi = idx_v[pl.ds(g * _L, _L)]
        m = (vi >= base) & (vi < base + _RPW)
        local = jnp.clip(vi - base, 0, _RPW - 1)
        pos = g * _L + iota
        cur = plsc.load_gather(win_v, [local], mask=m)
        plsc.store_scatter(win_v, [local], pos, mask=m & (pos > cur))

    def _collect(g, n):
        w = win_v[pl.ds(g * _L, _L)]
        keep = w >= 0
        rows = base + g * _L + iota
        plsc.store_compressed(rows_v.at[pl.ds(n, _L)], rows, mask=keep)
        plsc.store_compressed(pos_v.at[pl.ds(n, _L)],
                              jnp.clip(w, 0, _BATCH - 1), mask=keep)
        return n + jnp.max(plsc.all_reduce_population_count(keep))

    n = pl.loop(0, _NWV, init_carry=jnp.int32(0), unroll=2)(_collect)

    cnt_v[pl.ds(0, _L)] = jnp.broadcast_to(n, (_L,))
    pltpu.sync_copy(cnt_v, cnt_out.at[wid])

    def _g_start(c, kb, ob, sem):
        off = pl.multiple_of(c * _CH, _CH)
        rv = rows_v[pl.ds(off, _CH)]
        pv = pos_v[pl.ds(off, _CH)]
        pltpu.async_copy(k_in.at[pv], kb, sem)
        pltpu.async_copy(mem_in.at[rv], ob, sem)

    def _g_wait(c, kb, ob, sem):
        off = pl.multiple_of(c * _CH, _CH)
        rv = rows_v[pl.ds(off, _CH)]
        pv = pos_v[pl.ds(off, _CH)]
        pltpu.make_async_copy(k_in.at[pv], kb, sem).wait()
        pltpu.make_async_copy(mem_in.at[rv], ob, sem).wait()

    def _blend(kb, ob, sb):
        @pl.loop(0, _CH)
        def _rows(r):
            for t in range(_EMB // _L):
                sl = pl.ds(t * _L, _L)
                sb[r, sl] = (kb[r, sl] + ob[r, sl]) * _MOM

    def _s_start(c, sb, sem):
        off = pl.multiple_of(c * _CH, _CH)
        pv = pos_v[pl.ds(off, _CH)]
        pltpu.async_copy(sb, vals_out.at[pv], sem)

    def _s_wait(sb, sem):
        pv = pos_v[pl.ds(0, _CH)]
        pltpu.make_async_copy(sb, vals_out.at[pv], sem).wait()

    @pl.when(n > 0)
    def _update():
        zero16 = jnp.zeros((_L,), jnp.int32)
        r0 = plsc.load_gather(rows_v, [zero16])
        p0 = plsc.load_gather(pos_v, [zero16])
        for t in range(_CH // _L):
            rows_v[pl.ds(n + t * _L, _L)] = r0
            pos_v[pl.ds(n + t * _L, _L)] = p0

        pltpu.sync_copy(rows_v, rows_out.at[wid])
        pltpu.sync_copy(pos_v, pos_out.at[wid])

        nch = (n + _CH - 1) // _CH

        _g_start(0, kbuf0, oldbuf0, sem_g0)

        @pl.loop(0, nch, step=2)
        def _chunk2(c):
            @pl.when(c + 1 < nch)
            def _():
                _g_start(c + 1, kbuf1, oldbuf1, sem_g1)

            _g_wait(c, kbuf0, oldbuf0, sem_g0)

            @pl.when(c >= 2)
            def _():
                _s_wait(sbuf0, sem_s0)

            _blend(kbuf0, oldbuf0, sbuf0)
            _s_start(c, sbuf0, sem_s0)

            @pl.when(c + 2 < nch)
            def _():
                _g_start(c + 2, kbuf0, oldbuf0, sem_g0)

            @pl.when(c + 1 < nch)
            def _():
                _g_wait(c + 1, kbuf1, oldbuf1, sem_g1)

                @pl.when(c >= 1)
                def _():
                    _s_wait(sbuf1, sem_s1)

                _blend(kbuf1, oldbuf1, sbuf1)
                _s_start(c + 1, sbuf1, sem_s1)

        _s_wait(sbuf0, sem_s0)

        @pl.when(nch >= 2)
        def _():
            _s_wait(sbuf1, sem_s1)


def _body_b(vals_in, rows_in, pos_in, cnt_in, out_st,
            rows_v, pos_v, cnt_v, gb0, gb1,
            sem_g0, sem_g1, sem_s0, sem_s1):
    wid = _worker_id()

    pltpu.sync_copy(rows_in.at[wid], rows_v)
    pltpu.sync_copy(pos_in.at[wid], pos_v)
    pltpu.sync_copy(cnt_in.at[wid], cnt_v)
    n = jnp.max(cnt_v[pl.ds(0, _L)])

    def _g_start(c, gb, sem):
        off = pl.multiple_of(c * _CH, _CH)
        pv = pos_v[pl.ds(off, _CH)]
        pltpu.async_copy(vals_in.at[pv], gb, sem)

    def _g_wait(c, gb, sem):
        off = pl.multiple_of(c * _CH, _CH)
        pv = pos_v[pl.ds(off, _CH)]
        pltpu.make_async_copy(vals_in.at[pv], gb, sem).wait()

    def _s_start(c, gb, sem):
        off = pl.multiple_of(c * _CH, _CH)
        rv = rows_v[pl.ds(off, _CH)]
        pltpu.async_copy(gb, out_st.at[rv], sem)

    def _s_wait(gb, sem):
        rv = rows_v[pl.ds(0, _CH)]
        pltpu.make_async_copy(gb, out_st.at[rv], sem).wait()

    @pl.when(n > 0)
    def _scatter():
        nch = (n + _CH - 1) // _CH
        _g_start(0, gb0, sem_g0)

        @pl.loop(0, nch, step=2)
        def _chunk2(c):
            @pl.when(c + 1 < nch)
            def _():
                @pl.when(c >= 1)
                def _():
                    _s_wait(gb1, sem_s1)

                _g_start(c + 1, gb1, sem_g1)

            _g_wait(c, gb0, sem_g0)
            _s_start(c, gb0, sem_s0)

            @pl.when(c + 2 < nch)
            def _():
                _s_wait(gb0, sem_s0)
                _g_start(c + 2, gb0, sem_g0)

            @pl.when(c + 1 < nch)
            def _():
                _g_wait(c + 1, gb1, sem_g1)
                _s_start(c + 1, gb1, sem_s1)

        _s_wait(gb0, sem_s0)

        @pl.when(nch >= 2)
        def _():
            _s_wait(gb1, sem_s1)


_phase_a = pl.kernel(
    _body_a,
    out_type=(
        jax.ShapeDtypeStruct((_BATCH, _EMB), jnp.float32),
        jax.ShapeDtypeStruct((_NW, _PPAD), jnp.int32),
        jax.ShapeDtypeStruct((_NW, _PPAD), jnp.int32),
        jax.ShapeDtypeStruct((_NW, _L), jnp.int32),
    ),
    mesh=_MESH,
    compiler_params=_PARAMS,
    scratch_types=[
        pltpu.VMEM((_BATCH,), jnp.int32),
        pltpu.VMEM((_RPW,), jnp.int32),
        pltpu.VMEM((_PPAD,), jnp.int32),
        pltpu.VMEM((_PPAD,), jnp.int32),
        pltpu.VMEM((_L,), jnp.int32),
        pltpu.VMEM((_CH, _EMB), jnp.float32),
        pltpu.VMEM((_CH, _EMB), jnp.float32),
        pltpu.VMEM((_CH, _EMB), jnp.float32),
        pltpu.VMEM((_CH, _EMB), jnp.float32),
        pltpu.VMEM((_CH, _EMB), jnp.float32),
        pltpu.VMEM((_CH, _EMB), jnp.float32),
        pltpu.SemaphoreType.DMA,
        pltpu.SemaphoreType.DMA,
        pltpu.SemaphoreType.DMA,
        pltpu.SemaphoreType.DMA,
    ],
    name="feature_memory_vals",
)

_phase_b = pl.kernel(
    _body_b,
    out_type=(),
    mesh=_MESH,
    compiler_params=_PARAMS,
    scratch_types=[
        pltpu.VMEM((_PPAD,), jnp.int32),
        pltpu.VMEM((_PPAD,), jnp.int32),
        pltpu.VMEM((_L,), jnp.int32),
        pltpu.VMEM((_CH, _EMB), jnp.float32),
        pltpu.VMEM((_CH, _EMB), jnp.float32),
        pltpu.SemaphoreType.DMA,
        pltpu.SemaphoreType.DMA,
        pltpu.SemaphoreType.DMA,
        pltpu.SemaphoreType.DMA,
    ],
    name="feature_memory_scatter",
)


def kernel(memory, k, idx):
    out_ref = jax.new_ref(memory)
    vals, rows_m, pos_m, cnt_m = _phase_a(memory, k, idx)
    _phase_b(vals, rows_m, pos_m, cnt_m, out_ref)
    return jax.freeze(out_ref)


# speedup vs baseline: 1.0002x; 1.0002x over previous
import jax
import jax.numpy as jnp
from jax import lax
from jax.experimental import pallas as pl
from jax.experimental.pallas import tpu as pltpu
from jax.experimental.pallas import tpu_sc as plsc

_SIZE = 220000
_EMB = 256
_BATCH = 16384
_MOM = 0.5

_NC, _NS, _L = 2, 16, 16
_NW = _NC * _NS
_RPW = 6880
_PPAD = 6960
_CH = 16
_NIV = _BATCH // _L
_NWV = _RPW // _L

_MESH = plsc.VectorSubcoreMesh(core_axis_name="c", subcore_axis_name="s",
                               num_cores=_NC, num_subcores=_NS)
_PARAMS = pltpu.CompilerParams(needs_layout_passes=False)


def _worker_id():
    return lax.axis_index("c") * _NS + lax.axis_index("s")


def _body_a(mem_in, k_in, idx_in, vals_out, rows_out, pos_out, cnt_out,
            idx_v, win_v, rows_v, pos_v, cnt_v,
            kbuf0, kbuf1, oldbuf0, oldbuf1, sbuf0, sbuf1,
            sem_g0, sem_g1, sem_s0, sem_s1):
    wid = _worker_id()
    base = wid * _RPW

    pltpu.sync_copy(idx_in, idx_v)

    iota = lax.iota(jnp.int32, _L)
    neg1 = jnp.full((_L,), -1, jnp.int32)

    @pl.loop(0, _NWV, unroll=5)
    def _init(g):
        win_v[pl.ds(g * _L, _L)] = neg1

    @pl.loop(0, _NIV, unroll=4)
    def _scan(g):
        vi = idx_v[pl.ds(g * _L, _L)]
        m = (vi >= base) & (vi < base + _RPW)
        local = jnp.clip(vi - base, 0, _RPW - 1)
        pos = g * _L + iota
        plsc.store_scatter(win_v, [local], pos, mask=m)

    @pl.loop(0, _NIV, unroll=4)
    def _fix(g):
        vi = idx_v[pl.ds(g * _L, _L)]
        m = (vi >= base) & (vi < base + _RPW)
        local = jnp.clip(vi - base, 0, _RPW - 1)
        pos = g * _L + iota
        cur = plsc.load_gather(win_v, [local], mask=m)
        plsc.store_scatter(win_v, [local], pos, mask=m & (pos > cur))

    def _collect(g, n):
        w = win_v[pl.ds(g * _L, _L)]
        keep = w >= 0
        rows = base + g * _L + iota
        plsc.store_compressed(rows_v.at[pl.ds(n, _L)], rows, mask=keep)
        plsc.store_compressed(pos_v.at[pl.ds(n, _L)],
                              jnp.clip(w, 0, _BATCH - 1), mask=keep)
        return n + jnp.max(plsc.all_reduce_population_count(keep))

    n = pl.loop(0, _NWV, init_carry=jnp.int32(0), unroll=2)(_collect)

    cnt_v[pl.ds(0, _L)] = jnp.broadcast_to(n, (_L,))
    pltpu.sync_copy(cnt_v, cnt_out.at[wid])

    def _g_start(c, kb, ob, sem):
        off = pl.multiple_of(c * _CH, _CH)
        rv = rows_v[pl.ds(off, _CH)]
        pv = pos_v[pl.ds(off, _CH)]
        pltpu.async_copy(k_in.at[pv], kb, sem)
        pltpu.async_copy(mem_in.at[rv], ob, sem)

    def _g_wait(c, kb, ob, sem):
        off = pl.multiple_of(c * _CH, _CH)
        rv = rows_v[pl.ds(off, _CH)]
        pv = pos_v[pl.ds(off, _CH)]
        pltpu.make_async_copy(k_in.at[pv], kb, sem).wait()
        pltpu.make_async_copy(mem_in.at[rv], ob, sem).wait()

    def _blend(kb, ob, sb):
        @pl.loop(0, _CH)
        def _rows(r):
            for t in range(_EMB // _L):
                sl = pl.ds(t * _L, _L)
                sb[r, sl] = (kb[r, sl] + ob[r, sl]) * _MOM

    def _s_start(c, sb, sem):
        off = pl.multiple_of(c * _CH, _CH)
        pv = pos_v[pl.ds(off, _CH)]
        pltpu.async_copy(sb, vals_out.at[pv], sem)

    def _s_wait(sb, sem):
        pv = pos_v[pl.ds(0, _CH)]
        pltpu.make_async_copy(sb, vals_out.at[pv], sem).wait()

    @pl.when(n > 0)
    def _update():
        zero16 = jnp.zeros((_L,), jnp.int32)
        r0 = plsc.load_gather(rows_v, [zero16])
        p0 = plsc.load_gather(pos_v, [zero16])
        for t in range(_CH // _L):
            rows_v[pl.ds(n + t * _L, _L)] = r0
            pos_v[pl.ds(n + t * _L, _L)] = p0

        pltpu.sync_copy(rows_v, rows_out.at[wid])
        pltpu.sync_copy(pos_v, pos_out.at[wid])

        nch = (n + _CH - 1) // _CH

        _g_start(0, kbuf0, oldbuf0, sem_g0)

        @pl.loop(0, nch, step=2)
        def _chunk2(c):
            @pl.when(c + 1 < nch)
            def _():
                _g_start(c + 1, kbuf1, oldbuf1, sem_g1)

            _g_wait(c, kbuf0, oldbuf0, sem_g0)

            @pl.when(c >= 2)
            def _():
                _s_wait(sbuf0, sem_s0)

            _blend(kbuf0, oldbuf0, sbuf0)
            _s_start(c, sbuf0, sem_s0)

            @pl.when(c + 2 < nch)
            def _():
                _g_start(c + 2, kbuf0, oldbuf0, sem_g0)

            @pl.when(c + 1 < nch)
            def _():
                _g_wait(c + 1, kbuf1, oldbuf1, sem_g1)

                @pl.when(c >= 1)
                def _():
                    _s_wait(sbuf1, sem_s1)

                _blend(kbuf1, oldbuf1, sbuf1)
                _s_start(c + 1, sbuf1, sem_s1)

        _s_wait(sbuf0, sem_s0)

        @pl.when(nch >= 2)
        def _():
            _s_wait(sbuf1, sem_s1)


def _body_b(vals_in, rows_in, pos_in, cnt_in, out_st,
            rows_v, pos_v, cnt_v, gb0, gb1,
            sem_g0, sem_g1, sem_s0, sem_s1):
    wid = _worker_id()

    pltpu.sync_copy(rows_in.at[wid], rows_v)
    pltpu.sync_copy(pos_in.at[wid], pos_v)
    pltpu.sync_copy(cnt_in.at[wid], cnt_v)
    n = jnp.max(cnt_v[pl.ds(0, _L)])

    def _g_start(c, gb, sem):
        off = pl.multiple_of(c * _CH, _CH)
        pv = pos_v[pl.ds(off, _CH)]
        pltpu.async_copy(vals_in.at[pv], gb, sem)

    def _g_wait(c, gb, sem):
        off = pl.multiple_of(c * _CH, _CH)
        pv = pos_v[pl.ds(off, _CH)]
        pltpu.make_async_copy(vals_in.at[pv], gb, sem).wait()

    def _s_start(c, gb, sem):
        off = pl.multiple_of(c * _CH, _CH)
        rv = rows_v[pl.ds(off, _CH)]
        pltpu.async_copy(gb, out_st.at[rv], sem)

    def _s_wait(gb, sem):
        rv = rows_v[pl.ds(0, _CH)]
        pltpu.make_async_copy(gb, out_st.at[rv], sem).wait()

    @pl.when(n > 0)
    def _scatter():
        nch = (n + _CH - 1) // _CH
        _g_start(0, gb0, sem_g0)

        @pl.loop(0, nch, step=2)
        def _chunk2(c):
            @pl.when(c + 1 < nch)
            def _():
                @pl.when(c >= 1)
                def _():
                    _s_wait(gb1, sem_s1)

                _g_start(c + 1, gb1, sem_g1)

            _g_wait(c, gb0, sem_g0)
            _s_start(c, gb0, sem_s0)

            @pl.when(c + 2 < nch)
            def _():
                _s_wait(gb0, sem_s0)
                _g_start(c + 2, gb0, sem_g0)

            @pl.when(c + 1 < nch)
            def _():
                _g_wait(c + 1, gb1, sem_g1)
                _s_start(c + 1, gb1, sem_s1)

        _s_wait(gb0, sem_s0)

        @pl.when(nch >= 2)
        def _():
            _s_wait(gb1, sem_s1)


_phase_a = pl.kernel(
    _body_a,
    out_type=(
        jax.ShapeDtypeStruct((_BATCH, _EMB), jnp.float32),
        jax.ShapeDtypeStruct((_NW, _PPAD), jnp.int32),
        jax.ShapeDtypeStruct((_NW, _PPAD), jnp.int32),
        jax.ShapeDtypeStruct((_NW, _L), jnp.int32),
    ),
    mesh=_MESH,
    compiler_params=_PARAMS,
    scratch_types=[
        pltpu.VMEM((_BATCH,), jnp.int32),
        pltpu.VMEM((_RPW,), jnp.int32),
        pltpu.VMEM((_PPAD,), jnp.int32),
        pltpu.VMEM((_PPAD,), jnp.int32),
        pltpu.VMEM((_L,), jnp.int32),
        pltpu.VMEM((_CH, _EMB), jnp.float32),
        pltpu.VMEM((_CH, _EMB), jnp.float32),
        pltpu.VMEM((_CH, _EMB), jnp.float32),
        pltpu.VMEM((_CH, _EMB), jnp.float32),
        pltpu.VMEM((_CH, _EMB), jnp.float32),
        pltpu.VMEM((_CH, _EMB), jnp.float32),
        pltpu.SemaphoreType.DMA,
        pltpu.SemaphoreType.DMA,
        pltpu.SemaphoreType.DMA,
        pltpu.SemaphoreType.DMA,
    ],
    name="feature_memory_vals",
)

_phase_b = pl.kernel(
    _body_b,
    out_type=(),
    mesh=_MESH,
    compiler_params=_PARAMS,
    scratch_types=[
        pltpu.VMEM((_PPAD,), jnp.int32),
        pltpu.VMEM((_PPAD,), jnp.int32),
        pltpu.VMEM((_L,), jnp.int32),
        pltpu.VMEM((_CH, _EMB), jnp.float32),
        pltpu.VMEM((_CH, _EMB), jnp.float32),
        pltpu.SemaphoreType.DMA,
        pltpu.SemaphoreType.DMA,
        pltpu.SemaphoreType.DMA,
        pltpu.SemaphoreType.DMA,
    ],
    name="feature_memory_scatter",
)


def kernel(memory, k, idx):
    vals, rows_m, pos_m, cnt_m = _phase_a(memory, k, idx)
    out_ref = jax.new_ref(memory)
    _phase_b(vals, rows_m, pos_m, cnt_m, out_ref)
    return jax.freeze(out_ref)


# speedup vs baseline: 1.1510x; 1.1508x over previous
import jax
import jax.numpy as jnp
from jax import lax
from jax.experimental import pallas as pl
from jax.experimental.pallas import tpu as pltpu
from jax.experimental.pallas import tpu_sc as plsc

_SIZE = 220000
_EMB = 256
_BATCH = 16384
_MOM = 0.5

_NC, _NS, _L = 2, 16, 16
_NW = _NC * _NS
_RPW = 6880
_CPA = 6720
_PPAD = 6960
_CH = 32
_NIV = _BATCH // _L
_NWV = _RPW // _L


def _body(mem_in, k_in, idx_in, out_st,
          idx_v, win_v, rows_v, pos_v,
          kbuf0, kbuf1, oldbuf0, oldbuf1, sbuf0, sbuf1,
          sem_g0, sem_g1, sem_s0, sem_s1):
    cid = lax.axis_index("c")
    sid = lax.axis_index("s")
    wid = cid * _NS + sid
    base = wid * _RPW

    pltpu.sync_copy(idx_in, idx_v)

    iota = lax.iota(jnp.int32, _L)
    neg1 = jnp.full((_L,), -1, jnp.int32)

    @pl.loop(0, _NWV, unroll=5)
    def _init(g):
        win_v[pl.ds(g * _L, _L)] = neg1

    @pl.loop(0, _NIV, unroll=4)
    def _scan(g):
        vi = idx_v[pl.ds(g * _L, _L)]
        m = (vi >= base) & (vi < base + _RPW)
        local = jnp.clip(vi - base, 0, _RPW - 1)
        pos = g * _L + iota
        plsc.store_scatter(win_v, [local], pos, mask=m)

    @pl.loop(0, _NIV, unroll=4)
    def _fix(g):
        vi = idx_v[pl.ds(g * _L, _L)]
        m = (vi >= base) & (vi < base + _RPW)
        local = jnp.clip(vi - base, 0, _RPW - 1)
        pos = g * _L + iota
        cur = plsc.load_gather(win_v, [local], mask=m)
        plsc.store_scatter(win_v, [local], pos, mask=m & (pos > cur))

    def _collect(g, n):
        w = win_v[pl.ds(g * _L, _L)]
        keep = w >= 0
        rows = base + g * _L + iota
        plsc.store_compressed(rows_v.at[pl.ds(n, _L)], rows, mask=keep)
        plsc.store_compressed(pos_v.at[pl.ds(n, _L)],
                              jnp.clip(w, 0, _BATCH - 1), mask=keep)
        return n + jnp.max(plsc.all_reduce_population_count(keep))

    n = pl.loop(0, _NWV, init_carry=jnp.int32(0), unroll=2)(_collect)

    def _g_start(c, kb, ob, sem):
        off = pl.multiple_of(c * _CH, _CH)
        for q in range(_CH // _L):
            sl = pl.ds(q * _L, _L)
            rv = rows_v[pl.ds(off + q * _L, _L)]
            pv = pos_v[pl.ds(off + q * _L, _L)]
            pltpu.async_copy(k_in.at[pv], kb.at[sl, :], sem)
            pltpu.async_copy(mem_in.at[rv], ob.at[sl, :], sem)

    def _g_wait(c, kb, ob, sem):
        off = pl.multiple_of(c * _CH, _CH)
        for q in range(_CH // _L):
            sl = pl.ds(q * _L, _L)
            rv = rows_v[pl.ds(off + q * _L, _L)]
            pv = pos_v[pl.ds(off + q * _L, _L)]
            pltpu.make_async_copy(k_in.at[pv], kb.at[sl, :], sem).wait()
            pltpu.make_async_copy(mem_in.at[rv], ob.at[sl, :], sem).wait()

    def _blend(kb, ob, sb):
        @pl.loop(0, _CH)
        def _rows(r):
            for t in range(_EMB // _L):
                sl = pl.ds(t * _L, _L)
                sb[r, sl] = (kb[r, sl] + ob[r, sl]) * _MOM

    def _s_start(c, sb, sem):
        off = pl.multiple_of(c * _CH, _CH)
        for q in range(_CH // _L):
            sl = pl.ds(q * _L, _L)
            rv = rows_v[pl.ds(off + q * _L, _L)]
            pltpu.async_copy(sb.at[sl, :], out_st.at[rv], sem)

    def _s_wait(sb, sem):
        for q in range(_CH // _L):
            sl = pl.ds(q * _L, _L)
            rv = rows_v[pl.ds(q * _L, _L)]
            pltpu.make_async_copy(sb.at[sl, :], out_st.at[rv], sem).wait()

    @pl.when(n > 0)
    def _update():
        zero16 = jnp.zeros((_L,), jnp.int32)
        r0 = plsc.load_gather(rows_v, [zero16])
        p0 = plsc.load_gather(pos_v, [zero16])
        for t in range(_CH // _L):
            rows_v[pl.ds(n + t * _L, _L)] = r0
            pos_v[pl.ds(n + t * _L, _L)] = p0

        nch = (n + _CH - 1) // _CH

        _g_start(0, kbuf0, oldbuf0, sem_g0)

        @pl.loop(0, nch, step=2)
        def _chunk2(c):
            @pl.when(c + 1 < nch)
            def _():
                _g_start(c + 1, kbuf1, oldbuf1, sem_g1)

            _g_wait(c, kbuf0, oldbuf0, sem_g0)

            @pl.when(c >= 2)
            def _():
                _s_wait(sbuf0, sem_s0)

            _blend(kbuf0, oldbuf0, sbuf0)
            _s_start(c, sbuf0, sem_s0)

            @pl.when(c + 2 < nch)
            def _():
                _g_start(c + 2, kbuf0, oldbuf0, sem_g0)

            @pl.when(c + 1 < nch)
            def _():
                _g_wait(c + 1, kbuf1, oldbuf1, sem_g1)

                @pl.when(c >= 1)
                def _():
                    _s_wait(sbuf1, sem_s1)

                _blend(kbuf1, oldbuf1, sbuf1)
                _s_start(c + 1, sbuf1, sem_s1)

        _s_wait(sbuf0, sem_s0)

        @pl.when(nch >= 2)
        def _():
            _s_wait(sbuf1, sem_s1)


_sc_update = pl.kernel(
    _body,
    out_type=(),
    mesh=plsc.VectorSubcoreMesh(core_axis_name="c", subcore_axis_name="s",
                                num_cores=_NC, num_subcores=_NS),
    compiler_params=pltpu.CompilerParams(needs_layout_passes=False),
    scratch_types=[
        pltpu.VMEM((_BATCH,), jnp.int32),
        pltpu.VMEM((_RPW,), jnp.int32),
        pltpu.VMEM((_PPAD,), jnp.int32),
        pltpu.VMEM((_PPAD,), jnp.int32),
        pltpu.VMEM((_CH, _EMB), jnp.float32),
        pltpu.VMEM((_CH, _EMB), jnp.float32),
        pltpu.VMEM((_CH, _EMB), jnp.float32),
        pltpu.VMEM((_CH, _EMB), jnp.float32),
        pltpu.VMEM((_CH, _EMB), jnp.float32),
        pltpu.VMEM((_CH, _EMB), jnp.float32),
        pltpu.SemaphoreType.DMA,
        pltpu.SemaphoreType.DMA,
        pltpu.SemaphoreType.DMA,
        pltpu.SemaphoreType.DMA,
    ],
    name="feature_memory_update",
)


def kernel(memory, k, idx):
    out_ref = jax.new_ref(memory)
    _sc_update(memory, k, idx, out_ref)
    return jax.freeze(out_ref)


# speedup vs baseline: 1.1519x; 1.0008x over previous
import jax
import jax.numpy as jnp
from jax import lax
from jax.experimental import pallas as pl
from jax.experimental.pallas import tpu as pltpu
from jax.experimental.pallas import tpu_sc as plsc

_SIZE = 220000
_EMB = 256
_BATCH = 16384
_MOM = 0.5

_NC, _NS, _L = 2, 16, 16
_NW = _NC * _NS
_RPW = 6880
_CPA = 6720
_PPAD = 6960
_CH = 32
_NIV = _BATCH // _L
_NWV = _RPW // _L


def _body(mem_in, k_in, idx_in, out_st,
          idx_v, win_v, rows_v, pos_v,
          kbuf0, kbuf1, oldbuf0, oldbuf1, sbuf0, sbuf1,
          sem_g0, sem_g1, sem_s0, sem_s1):
    cid = lax.axis_index("c")
    sid = lax.axis_index("s")
    wid = cid * _NS + sid
    base = wid * _RPW

    pltpu.sync_copy(idx_in, idx_v)

    iota = lax.iota(jnp.int32, _L)
    neg1 = jnp.full((_L,), -1, jnp.int32)

    @pl.loop(0, _NWV, unroll=5)
    def _init(g):
        win_v[pl.ds(g * _L, _L)] = neg1

    @pl.loop(0, _NIV, unroll=8)
    def _scan(g):
        vi = idx_v[pl.ds(g * _L, _L)]
        m = (vi >= base) & (vi < base + _RPW)
        local = jnp.clip(vi - base, 0, _RPW - 1)
        pos = g * _L + iota
        plsc.store_scatter(win_v, [local], pos, mask=m)

    @pl.loop(0, _NIV, unroll=8)
    def _fix(g):
        vi = idx_v[pl.ds(g * _L, _L)]
        m = (vi >= base) & (vi < base + _RPW)
        local = jnp.clip(vi - base, 0, _RPW - 1)
        pos = g * _L + iota
        cur = plsc.load_gather(win_v, [local], mask=m)
        plsc.store_scatter(win_v, [local], pos, mask=m & (pos > cur))

    def _collect(g, n):
        w = win_v[pl.ds(g * _L, _L)]
        keep = w >= 0
        rows = base + g * _L + iota
        plsc.store_compressed(rows_v.at[pl.ds(n, _L)], rows, mask=keep)
        plsc.store_compressed(pos_v.at[pl.ds(n, _L)],
                              jnp.clip(w, 0, _BATCH - 1), mask=keep)
        return n + jnp.max(plsc.all_reduce_population_count(keep))

    n = pl.loop(0, _NWV, init_carry=jnp.int32(0), unroll=5)(_collect)

    def _g_start(c, kb, ob, sem):
        off = pl.multiple_of(c * _CH, _CH)
        for q in range(_CH // _L):
            sl = pl.ds(q * _L, _L)
            rv = rows_v[pl.ds(off + q * _L, _L)]
            pv = pos_v[pl.ds(off + q * _L, _L)]
            pltpu.async_copy(k_in.at[pv], kb.at[sl, :], sem)
            pltpu.async_copy(mem_in.at[rv], ob.at[sl, :], sem)

    def _g_wait(c, kb, ob, sem):
        off = pl.multiple_of(c * _CH, _CH)
        for q in range(_CH // _L):
            sl = pl.ds(q * _L, _L)
            rv = rows_v[pl.ds(off + q * _L, _L)]
            pv = pos_v[pl.ds(off + q * _L, _L)]
            pltpu.make_async_copy(k_in.at[pv], kb.at[sl, :], sem).wait()
            pltpu.make_async_copy(mem_in.at[rv], ob.at[sl, :], sem).wait()

    def _blend(kb, ob, sb):
        @pl.loop(0, _CH)
        def _rows(r):
            for t in range(_EMB // _L):
                sl = pl.ds(t * _L, _L)
                sb[r, sl] = (kb[r, sl] + ob[r, sl]) * _MOM

    def _s_start(c, sb, sem):
        off = pl.multiple_of(c * _CH, _CH)
        for q in range(_CH // _L):
            sl = pl.ds(q * _L, _L)
            rv = rows_v[pl.ds(off + q * _L, _L)]
            pltpu.async_copy(sb.at[sl, :], out_st.at[rv], sem)

    def _s_wait(sb, sem):
        for q in range(_CH // _L):
            sl = pl.ds(q * _L, _L)
            rv = rows_v[pl.ds(q * _L, _L)]
            pltpu.make_async_copy(sb.at[sl, :], out_st.at[rv], sem).wait()

    @pl.when(n > 0)
    def _update():
        zero16 = jnp.zeros((_L,), jnp.int32)
        r0 = plsc.load_gather(rows_v, [zero16])
        p0 = plsc.load_gather(pos_v, [zero16])
        for t in range(_CH // _L):
            rows_v[pl.ds(n + t * _L, _L)] = r0
            pos_v[pl.ds(n + t * _L, _L)] = p0

        nch = (n + _CH - 1) // _CH

        _g_start(0, kbuf0, oldbuf0, sem_g0)

        @pl.loop(0, nch, step=2)
        def _chunk2(c):
            @pl.when(c + 1 < nch)
            def _():
                _g_start(c + 1, kbuf1, oldbuf1, sem_g1)

            _g_wait(c, kbuf0, oldbuf0, sem_g0)

            @pl.when(c >= 2)
            def _():
                _s_wait(sbuf0, sem_s0)

            _blend(kbuf0, oldbuf0, sbuf0)
            _s_start(c, sbuf0, sem_s0)

            @pl.when(c + 2 < nch)
            def _():
                _g_start(c + 2, kbuf0, oldbuf0, sem_g0)

            @pl.when(c + 1 < nch)
            def _():
                _g_wait(c + 1, kbuf1, oldbuf1, sem_g1)

                @pl.when(c >= 1)
                def _():
                    _s_wait(sbuf1, sem_s1)

                _blend(kbuf1, oldbuf1, sbuf1)
                _s_start(c + 1, sbuf1, sem_s1)

        _s_wait(sbuf0, sem_s0)

        @pl.when(nch >= 2)
        def _():
            _s_wait(sbuf1, sem_s1)


_sc_update = pl.kernel(
    _body,
    out_type=(),
    mesh=plsc.VectorSubcoreMesh(core_axis_name="c", subcore_axis_name="s",
                                num_cores=_NC, num_subcores=_NS),
    compiler_params=pltpu.CompilerParams(needs_layout_passes=False),
    scratch_types=[
        pltpu.VMEM((_BATCH,), jnp.int32),
        pltpu.VMEM((_RPW,), jnp.int32),
        pltpu.VMEM((_PPAD,), jnp.int32),
        pltpu.VMEM((_PPAD,), jnp.int32),
        pltpu.VMEM((_CH, _EMB), jnp.float32),
        pltpu.VMEM((_CH, _EMB), jnp.float32),
        pltpu.VMEM((_CH, _EMB), jnp.float32),
        pltpu.VMEM((_CH, _EMB), jnp.float32),
        pltpu.VMEM((_CH, _EMB), jnp.float32),
        pltpu.VMEM((_CH, _EMB), jnp.float32),
        pltpu.SemaphoreType.DMA,
        pltpu.SemaphoreType.DMA,
        pltpu.SemaphoreType.DMA,
        pltpu.SemaphoreType.DMA,
    ],
    name="feature_memory_update",
)


def kernel(memory, k, idx):
    out_ref = jax.new_ref(memory)
    _sc_update(memory, k, idx, out_ref)
    return jax.freeze(out_ref)


# speedup vs baseline: 1.1689x; 1.0148x over previous
import jax
import jax.numpy as jnp
from jax import lax
from jax.experimental import pallas as pl
from jax.experimental.pallas import tpu as pltpu
from jax.experimental.pallas import tpu_sc as plsc

_SIZE = 220000
_EMB = 256
_BATCH = 16384
_MOM = 0.5

_NC, _NS, _L = 2, 16, 16
_NW = _NC * _NS
_RPW = 6880
_CPA = 6720
_PPAD = 6960
_CH = 32
_NIV = _BATCH // _L
_NWV = _RPW // _L


def _body(mem_in, k_in, idx_in, out_st,
          idx_v, win_v, rows_v, pos_v,
          kbuf0, kbuf1, kbuf2, oldbuf0, oldbuf1, oldbuf2,
          sbuf0, sbuf1, sbuf2,
          sem_g0, sem_g1, sem_g2, sem_s0, sem_s1, sem_s2):
    cid = lax.axis_index("c")
    sid = lax.axis_index("s")
    wid = cid * _NS + sid
    base = wid * _RPW

    pltpu.sync_copy(idx_in, idx_v)

    iota = lax.iota(jnp.int32, _L)
    neg1 = jnp.full((_L,), -1, jnp.int32)

    @pl.loop(0, _NWV, unroll=5)
    def _init(g):
        win_v[pl.ds(g * _L, _L)] = neg1

    @pl.loop(0, _NIV, unroll=8)
    def _scan(g):
        vi = idx_v[pl.ds(g * _L, _L)]
        m = (vi >= base) & (vi < base + _RPW)
        local = jnp.clip(vi - base, 0, _RPW - 1)
        pos = g * _L + iota
        plsc.store_scatter(win_v, [local], pos, mask=m)

    @pl.loop(0, _NIV, unroll=8)
    def _fix(g):
        vi = idx_v[pl.ds(g * _L, _L)]
        m = (vi >= base) & (vi < base + _RPW)
        local = jnp.clip(vi - base, 0, _RPW - 1)
        pos = g * _L + iota
        cur = plsc.load_gather(win_v, [local], mask=m)
        plsc.store_scatter(win_v, [local], pos, mask=m & (pos > cur))

    def _collect(g, n):
        w = win_v[pl.ds(g * _L, _L)]
        keep = w >= 0
        rows = base + g * _L + iota
        plsc.store_compressed(rows_v.at[pl.ds(n, _L)], rows, mask=keep)
        plsc.store_compressed(pos_v.at[pl.ds(n, _L)],
                              jnp.clip(w, 0, _BATCH - 1), mask=keep)
        return n + jnp.max(plsc.all_reduce_population_count(keep))

    n = pl.loop(0, _NWV, init_carry=jnp.int32(0), unroll=5)(_collect)

    def _g_start(c, kb, ob, sem):
        off = pl.multiple_of(c * _CH, _CH)
        for q in range(_CH // _L):
            sl = pl.ds(q * _L, _L)
            rv = rows_v[pl.ds(off + q * _L, _L)]
            pv = pos_v[pl.ds(off + q * _L, _L)]
            pltpu.async_copy(k_in.at[pv], kb.at[sl, :], sem)
            pltpu.async_copy(mem_in.at[rv], ob.at[sl, :], sem)

    def _g_wait(c, kb, ob, sem):
        off = pl.multiple_of(c * _CH, _CH)
        for q in range(_CH // _L):
            sl = pl.ds(q * _L, _L)
            rv = rows_v[pl.ds(off + q * _L, _L)]
            pv = pos_v[pl.ds(off + q * _L, _L)]
            pltpu.make_async_copy(k_in.at[pv], kb.at[sl, :], sem).wait()
            pltpu.make_async_copy(mem_in.at[rv], ob.at[sl, :], sem).wait()

    def _blend(kb, ob, sb):
        @pl.loop(0, _CH)
        def _rows(r):
            for t in range(_EMB // _L):
                sl = pl.ds(t * _L, _L)
                sb[r, sl] = (kb[r, sl] + ob[r, sl]) * _MOM

    def _s_start(c, sb, sem):
        off = pl.multiple_of(c * _CH, _CH)
        for q in range(_CH // _L):
            sl = pl.ds(q * _L, _L)
            rv = rows_v[pl.ds(off + q * _L, _L)]
            pltpu.async_copy(sb.at[sl, :], out_st.at[rv], sem)

    def _s_wait(sb, sem):
        for q in range(_CH // _L):
            sl = pl.ds(q * _L, _L)
            rv = rows_v[pl.ds(q * _L, _L)]
            pltpu.make_async_copy(sb.at[sl, :], out_st.at[rv], sem).wait()

    @pl.when(n > 0)
    def _update():
        zero16 = jnp.zeros((_L,), jnp.int32)
        r0 = plsc.load_gather(rows_v, [zero16])
        p0 = plsc.load_gather(pos_v, [zero16])
        for t in range(_CH // _L):
            rows_v[pl.ds(n + t * _L, _L)] = r0
            pos_v[pl.ds(n + t * _L, _L)] = p0

        nch = (n + _CH - 1) // _CH

        sets = ((kbuf0, oldbuf0, sbuf0, sem_g0, sem_s0),
                (kbuf1, oldbuf1, sbuf1, sem_g1, sem_s1),
                (kbuf2, oldbuf2, sbuf2, sem_g2, sem_s2))

        _g_start(0, kbuf0, oldbuf0, sem_g0)

        @pl.when(1 < nch)
        def _():
            _g_start(1, kbuf1, oldbuf1, sem_g1)

        @pl.when(2 < nch)
        def _():
            _g_start(2, kbuf2, oldbuf2, sem_g2)

        @pl.loop(0, nch, step=3)
        def _chunk3(c):
            for j in range(3):
                kb, ob, sb, sg, ss = sets[j]
                cc = c + j

                @pl.when(cc < nch)
                def _():
                    _g_wait(cc, kb, ob, sg)

                    @pl.when(cc >= 3)
                    def _():
                        _s_wait(sb, ss)

                    _blend(kb, ob, sb)
                    _s_start(cc, sb, ss)

                    @pl.when(cc + 3 < nch)
                    def _():
                        _g_start(cc + 3, kb, ob, sg)

        for j in range(3):
            kb, ob, sb, sg, ss = sets[j]

            @pl.when(nch > j)
            def _():
                _s_wait(sb, ss)


_sc_update = pl.kernel(
    _body,
    out_type=(),
    mesh=plsc.VectorSubcoreMesh(core_axis_name="c", subcore_axis_name="s",
                                num_cores=_NC, num_subcores=_NS),
    compiler_params=pltpu.CompilerParams(needs_layout_passes=False),
    scratch_types=[
        pltpu.VMEM((_BATCH,), jnp.int32),
        pltpu.VMEM((_RPW,), jnp.int32),
        pltpu.VMEM((_PPAD,), jnp.int32),
        pltpu.VMEM((_PPAD,), jnp.int32),
        pltpu.VMEM((_CH, _EMB), jnp.float32),
        pltpu.VMEM((_CH, _EMB), jnp.float32),
        pltpu.VMEM((_CH, _EMB), jnp.float32),
        pltpu.VMEM((_CH, _EMB), jnp.float32),
        pltpu.VMEM((_CH, _EMB), jnp.float32),
        pltpu.VMEM((_CH, _EMB), jnp.float32),
        pltpu.VMEM((_CH, _EMB), jnp.float32),
        pltpu.VMEM((_CH, _EMB), jnp.float32),
        pltpu.VMEM((_CH, _EMB), jnp.float32),
        pltpu.SemaphoreType.DMA,
        pltpu.SemaphoreType.DMA,
        pltpu.SemaphoreType.DMA,
        pltpu.SemaphoreType.DMA,
        pltpu.SemaphoreType.DMA,
        pltpu.SemaphoreType.DMA,
    ],
    name="feature_memory_update",
)


def kernel(memory, k, idx):
    out_ref = jax.new_ref(memory)
    _sc_update(memory, k, idx, out_ref)
    return jax.freeze(out_ref)
